# Initial kernel scaffold; baseline (speedup 1.0000x reference)
import jax
import jax.numpy as jnp
from jax.experimental import pallas as pl


def kernel(inputs, adjacency_matrix, W1, a_src1, a_dst1, W2, a_src2, a_dst2):
    raise NotImplementedError("write your pallas kernel here")


# speedup vs baseline: 1.3713x; 1.3713x over previous
import functools

import jax
import jax.numpy as jnp
from jax.experimental import pallas as pl
from jax.experimental.pallas import tpu as pltpu

N = 4096
D = 256
BLK = 256


def _gat_layer_kernel(x_ref, w_ref, asrc_ref, adst_ref, adj_ref, out_ref, h_ref):
    i = pl.program_id(0)

    @pl.when(i == 0)
    def _():
        h_ref[...] = jnp.dot(x_ref[...], w_ref[...],
                             preferred_element_type=jnp.float32)

    h = h_ref[...]
    h_blk = h_ref[pl.ds(i * BLK, BLK), :]
    s = jnp.dot(h_blk, asrc_ref[...], preferred_element_type=jnp.float32)
    d_t = jax.lax.dot_general(
        adst_ref[...], h,
        dimension_numbers=(((0,), (1,)), ((), ())),
        preferred_element_type=jnp.float32)
    e = s + d_t
    e = jnp.where(e >= 0, e, 0.2 * e)
    e = jnp.where(adj_ref[...] > 0, e, jnp.float32(-1e9))
    m = jnp.max(e, axis=1, keepdims=True)
    p = jnp.exp(e - m)
    z = jnp.sum(p, axis=1, keepdims=True)
    agg = jnp.dot(p, h, preferred_element_type=jnp.float32) / z
    out_ref[...] = jnp.where(agg > 0, agg, jnp.exp(agg) - 1.0)


def _gat_layer(x, adj, W, a_src, a_dst):
    grid = (N // BLK,)
    return pl.pallas_call(
        _gat_layer_kernel,
        grid=grid,
        in_specs=[
            pl.BlockSpec((N, D), lambda i: (0, 0)),
            pl.BlockSpec((D, D), lambda i: (0, 0)),
            pl.BlockSpec((D, 1), lambda i: (0, 0)),
            pl.BlockSpec((D, 1), lambda i: (0, 0)),
            pl.BlockSpec((BLK, N), lambda i: (i, 0)),
        ],
        out_specs=pl.BlockSpec((BLK, D), lambda i: (i, 0)),
        out_shape=jax.ShapeDtypeStruct((N, D), jnp.float32),
        scratch_shapes=[pltpu.VMEM((N, D), jnp.float32)],
    )(x, W, a_src, a_dst, adj)


@jax.jit
def kernel(inputs, adjacency_matrix, W1, a_src1, a_dst1, W2, a_src2, a_dst2):
    x = _gat_layer(inputs, adjacency_matrix, W1, a_src1, a_dst1)
    x = _gat_layer(x, adjacency_matrix, W2, a_src2, a_dst2)
    return x


# speedup vs baseline: 1.6886x; 1.2314x over previous
import functools

import jax
import jax.numpy as jnp
from jax.experimental import pallas as pl
from jax.experimental.pallas import tpu as pltpu

N = 4096
D = 256
BLK = 256


def _gat_layer_kernel(x_ref, w_ref, asrc_ref, adst_ref, adj_ref, out_ref,
                      h_ref, hb_ref, dt_ref):
    i = pl.program_id(0)

    @pl.when(i == 0)
    def _():
        h = jnp.dot(x_ref[...], w_ref[...], preferred_element_type=jnp.float32)
        h_ref[...] = h
        hb_ref[...] = h.astype(jnp.bfloat16)
        d_t = jax.lax.dot_general(
            adst_ref[...], h,
            dimension_numbers=(((0,), (1,)), ((), ())),
            preferred_element_type=jnp.float32)
        dt_ref[...] = jnp.broadcast_to(d_t, (8, N))

    h_blk = h_ref[pl.ds(i * BLK, BLK), :]
    s = jnp.dot(h_blk, asrc_ref[...], preferred_element_type=jnp.float32)
    e = s + dt_ref[0:1, :]
    e = jnp.where(e >= 0, e, 0.2 * e)
    e = jnp.where(adj_ref[...] > 0, e, jnp.float32(-1e9))
    m = jnp.max(e, axis=1, keepdims=True)
    p = jnp.exp(e - m)
    z = jnp.sum(p, axis=1, keepdims=True)
    agg = jnp.dot(p.astype(jnp.bfloat16), hb_ref[...],
                  preferred_element_type=jnp.float32) / z
    out_ref[...] = jnp.where(agg > 0, agg, jnp.exp(agg) - 1.0)


def _gat_layer(x, adj, W, a_src, a_dst):
    grid = (N // BLK,)
    return pl.pallas_call(
        _gat_layer_kernel,
        grid=grid,
        in_specs=[
            pl.BlockSpec((N, D), lambda i: (0, 0)),
            pl.BlockSpec((D, D), lambda i: (0, 0)),
            pl.BlockSpec((D, 1), lambda i: (0, 0)),
            pl.BlockSpec((D, 1), lambda i: (0, 0)),
            pl.BlockSpec((BLK, N), lambda i: (i, 0)),
        ],
        out_specs=pl.BlockSpec((BLK, D), lambda i: (i, 0)),
        out_shape=jax.ShapeDtypeStruct((N, D), jnp.float32),
        scratch_shapes=[pltpu.VMEM((N, D), jnp.float32),
                        pltpu.VMEM((N, D), jnp.bfloat16),
                        pltpu.VMEM((8, N), jnp.float32)],
    )(x, W, a_src, a_dst, adj)


@jax.jit
def kernel(inputs, adjacency_matrix, W1, a_src1, a_dst1, W2, a_src2, a_dst2):
    x = _gat_layer(inputs, adjacency_matrix, W1, a_src1, a_dst1)
    x = _gat_layer(x, adjacency_matrix, W2, a_src2, a_dst2)
    return x


# speedup vs baseline: 1.9258x; 1.1404x over previous
import functools

import jax
import jax.numpy as jnp
from jax.experimental import pallas as pl
from jax.experimental.pallas import tpu as pltpu

N = 4096
D = 256
BLK = 512


def _gat_layer_kernel(x_ref, w_ref, asrc_ref, adst_ref, adj_ref, out_ref,
                      h_ref, hb_ref, dt_ref):
    i = pl.program_id(0)

    @pl.when(i == 0)
    def _():
        h = jnp.dot(x_ref[...], w_ref[...], preferred_element_type=jnp.float32)
        h_ref[...] = h
        hb_ref[...] = h.astype(jnp.bfloat16)
        d_t = jax.lax.dot_general(
            adst_ref[...], h,
            dimension_numbers=(((0,), (1,)), ((), ())),
            preferred_element_type=jnp.float32)
        dt_ref[...] = jnp.broadcast_to(d_t, (8, N))

    h_blk = h_ref[pl.ds(i * BLK, BLK), :]
    s = jnp.dot(h_blk, asrc_ref[...], preferred_element_type=jnp.float32)
    e = s + dt_ref[0:1, :]
    e = jnp.where(e >= 0, e, 0.2 * e)
    p = adj_ref[...] * jnp.exp(jnp.minimum(e, 60.0))
    z = jnp.sum(p, axis=1, keepdims=True)
    agg = jnp.dot(p.astype(jnp.bfloat16), hb_ref[...],
                  preferred_element_type=jnp.float32) / z
    out_ref[...] = jnp.where(agg > 0, agg, jnp.exp(agg) - 1.0)


def _gat_layer(x, adj, W, a_src, a_dst):
    grid = (N // BLK,)
    return pl.pallas_call(
        _gat_layer_kernel,
        grid=grid,
        in_specs=[
            pl.BlockSpec((N, D), lambda i: (0, 0)),
            pl.BlockSpec((D, D), lambda i: (0, 0)),
            pl.BlockSpec((D, 1), lambda i: (0, 0)),
            pl.BlockSpec((D, 1), lambda i: (0, 0)),
            pl.BlockSpec((BLK, N), lambda i: (i, 0)),
        ],
        out_specs=pl.BlockSpec((BLK, D), lambda i: (i, 0)),
        out_shape=jax.ShapeDtypeStruct((N, D), jnp.float32),
        scratch_shapes=[pltpu.VMEM((N, D), jnp.float32),
                        pltpu.VMEM((N, D), jnp.bfloat16),
                        pltpu.VMEM((8, N), jnp.float32)],
    )(x, W, a_src, a_dst, adj)


@jax.jit
def kernel(inputs, adjacency_matrix, W1, a_src1, a_dst1, W2, a_src2, a_dst2):
    x = _gat_layer(inputs, adjacency_matrix, W1, a_src1, a_dst1)
    x = _gat_layer(x, adjacency_matrix, W2, a_src2, a_dst2)
    return x


# speedup vs baseline: 2.0767x; 1.0784x over previous
import functools

import jax
import jax.numpy as jnp
from jax.experimental import pallas as pl
from jax.experimental.pallas import tpu as pltpu

N = 4096
D = 256
BLK = 512
CHUNK = 1024


def _gat_layer_kernel(x_ref, w_ref, asrc_ref, adst_ref, adj_ref, out_ref,
                      h_ref, hb_ref, dt_ref):
    i = pl.program_id(0)

    @pl.when(i == 0)
    def _():
        h = jnp.dot(x_ref[...], w_ref[...], preferred_element_type=jnp.float32)
        h_ref[...] = h
        hb_ref[...] = h.astype(jnp.bfloat16)
        d_t = jax.lax.dot_general(
            adst_ref[...], h,
            dimension_numbers=(((0,), (1,)), ((), ())),
            preferred_element_type=jnp.float32)
        dt_ref[...] = jnp.broadcast_to(d_t, (8, N))

    h_blk = h_ref[pl.ds(i * BLK, BLK), :]
    s = jnp.dot(h_blk, asrc_ref[...], preferred_element_type=jnp.float32)
    agg = jnp.zeros((BLK, D), jnp.float32)
    z = jnp.zeros((BLK, 1), jnp.float32)
    for c in range(N // CHUNK):
        lo = c * CHUNK
        e = s + dt_ref[0:1, pl.ds(lo, CHUNK)]
        e = jnp.maximum(e, 0.2 * e)
        p = adj_ref[:, pl.ds(lo, CHUNK)] * jnp.exp(jnp.minimum(e, 60.0))
        z = z + jnp.sum(p, axis=1, keepdims=True)
        agg = agg + jnp.dot(p.astype(jnp.bfloat16), hb_ref[pl.ds(lo, CHUNK), :],
                            preferred_element_type=jnp.float32)
    agg = agg / z
    out_ref[...] = jnp.where(agg > 0, agg, jnp.exp(agg) - 1.0)


def _gat_layer(x, adj, W, a_src, a_dst):
    grid = (N // BLK,)
    return pl.pallas_call(
        _gat_layer_kernel,
        grid=grid,
        in_specs=[
            pl.BlockSpec((N, D), lambda i: (0, 0)),
            pl.BlockSpec((D, D), lambda i: (0, 0)),
            pl.BlockSpec((D, 1), lambda i: (0, 0)),
            pl.BlockSpec((D, 1), lambda i: (0, 0)),
            pl.BlockSpec((BLK, N), lambda i: (i, 0)),
        ],
        out_specs=pl.BlockSpec((BLK, D), lambda i: (i, 0)),
        out_shape=jax.ShapeDtypeStruct((N, D), jnp.float32),
        scratch_shapes=[pltpu.VMEM((N, D), jnp.float32),
                        pltpu.VMEM((N, D), jnp.bfloat16),
                        pltpu.VMEM((8, N), jnp.float32)],
    )(x, W, a_src, a_dst, adj)


@jax.jit
def kernel(inputs, adjacency_matrix, W1, a_src1, a_dst1, W2, a_src2, a_dst2):
    x = _gat_layer(inputs, adjacency_matrix, W1, a_src1, a_dst1)
    x = _gat_layer(x, adjacency_matrix, W2, a_src2, a_dst2)
    return x


# speedup vs baseline: 2.2123x; 1.0653x over previous
import functools

import jax
import jax.numpy as jnp
from jax.experimental import pallas as pl
from jax.experimental.pallas import tpu as pltpu

N = 4096
D = 256
BLK = 1024
CHUNK = 512


def _gat_layer_kernel(x_ref, w_ref, asrc_ref, adst_ref, adj_ref, out_ref,
                      h_ref, hb_ref, dt_ref):
    i = pl.program_id(0)

    @pl.when(i == 0)
    def _():
        h = jnp.dot(x_ref[...], w_ref[...], preferred_element_type=jnp.float32)
        h_ref[...] = h
        hb_ref[...] = h.astype(jnp.bfloat16)
        d_t = jax.lax.dot_general(
            adst_ref[...], h,
            dimension_numbers=(((0,), (1,)), ((), ())),
            preferred_element_type=jnp.float32)
        dt_ref[...] = jnp.broadcast_to(jnp.minimum(d_t, 40.0), (8, N))

    h_blk = h_ref[pl.ds(i * BLK, BLK), :]
    s = jnp.minimum(
        jnp.dot(h_blk, asrc_ref[...], preferred_element_type=jnp.float32),
        40.0)
    agg = jnp.zeros((BLK, D), jnp.float32)
    z = jnp.zeros((BLK, 1), jnp.float32)
    for c in range(N // CHUNK):
        lo = c * CHUNK
        e = s + dt_ref[0:1, pl.ds(lo, CHUNK)]
        e = jnp.maximum(e, 0.2 * e)
        p = adj_ref[:, pl.ds(lo, CHUNK)] * jnp.exp(e)
        z = z + jnp.sum(p, axis=1, keepdims=True)
        agg = agg + jnp.dot(p.astype(jnp.bfloat16), hb_ref[pl.ds(lo, CHUNK), :],
                            preferred_element_type=jnp.float32)
    agg = agg / z
    out_ref[...] = jnp.where(agg > 0, agg, jnp.exp(agg) - 1.0)


def _gat_layer(x, adj, W, a_src, a_dst):
    grid = (N // BLK,)
    return pl.pallas_call(
        _gat_layer_kernel,
        grid=grid,
        in_specs=[
            pl.BlockSpec((N, D), lambda i: (0, 0)),
            pl.BlockSpec((D, D), lambda i: (0, 0)),
            pl.BlockSpec((D, 1), lambda i: (0, 0)),
            pl.BlockSpec((D, 1), lambda i: (0, 0)),
            pl.BlockSpec((BLK, N), lambda i: (i, 0)),
        ],
        out_specs=pl.BlockSpec((BLK, D), lambda i: (i, 0)),
        out_shape=jax.ShapeDtypeStruct((N, D), jnp.float32),
        scratch_shapes=[pltpu.VMEM((N, D), jnp.float32),
                        pltpu.VMEM((N, D), jnp.bfloat16),
                        pltpu.VMEM((8, N), jnp.float32)],
    )(x, W, a_src, a_dst, adj)


@jax.jit
def kernel(inputs, adjacency_matrix, W1, a_src1, a_dst1, W2, a_src2, a_dst2):
    x = _gat_layer(inputs, adjacency_matrix, W1, a_src1, a_dst1)
    x = _gat_layer(x, adjacency_matrix, W2, a_src2, a_dst2)
    return x


# speedup vs baseline: 2.2903x; 1.0352x over previous
import functools

import jax
import jax.numpy as jnp
from jax.experimental import pallas as pl
from jax.experimental.pallas import tpu as pltpu

N = 4096
D = 256
BLK = 512
DE = D + 128
CHUNK = 512


def _gat_layer_kernel(x_ref, w_ref, asrc_ref, adst_ref, adj_ref, out_ref,
                      h_ref, hb_ref, dt_ref):
    i = pl.program_id(0)

    @pl.when(i == 0)
    def _():
        h = jnp.dot(x_ref[...], w_ref[...], preferred_element_type=jnp.float32)
        h_ref[...] = h
        hb_ref[:, :D] = h.astype(jnp.bfloat16)
        hb_ref[:, D:] = jnp.ones((N, DE - D), jnp.bfloat16)
        d_t = jax.lax.dot_general(
            adst_ref[...], h,
            dimension_numbers=(((0,), (1,)), ((), ())),
            preferred_element_type=jnp.float32)
        dt_ref[...] = jnp.broadcast_to(jnp.minimum(d_t, 40.0), (8, N))

    h_blk = h_ref[pl.ds(i * BLK, BLK), :]
    s = jnp.minimum(
        jnp.dot(h_blk, asrc_ref[...], preferred_element_type=jnp.float32),
        40.0)
    acc = jnp.zeros((BLK, DE), jnp.float32)
    for c in range(N // CHUNK):
        lo = c * CHUNK
        e = s + dt_ref[0:1, pl.ds(lo, CHUNK)]
        e = jnp.maximum(e, 0.2 * e)
        p = adj_ref[:, pl.ds(lo, CHUNK)] * jnp.exp(e)
        acc = acc + jnp.dot(p.astype(jnp.bfloat16), hb_ref[pl.ds(lo, CHUNK), :],
                            preferred_element_type=jnp.float32)
    agg = acc[:, :D] / acc[:, D:D + 1]
    out_ref[...] = jnp.where(agg > 0, agg, jnp.exp(agg) - 1.0)


def _gat_layer(x, adj, W, a_src, a_dst):
    grid = (N // BLK,)
    return pl.pallas_call(
        _gat_layer_kernel,
        grid=grid,
        in_specs=[
            pl.BlockSpec((N, D), lambda i: (0, 0)),
            pl.BlockSpec((D, D), lambda i: (0, 0)),
            pl.BlockSpec((D, 1), lambda i: (0, 0)),
            pl.BlockSpec((D, 1), lambda i: (0, 0)),
            pl.BlockSpec((BLK, N), lambda i: (i, 0)),
        ],
        out_specs=pl.BlockSpec((BLK, D), lambda i: (i, 0)),
        out_shape=jax.ShapeDtypeStruct((N, D), jnp.float32),
        scratch_shapes=[pltpu.VMEM((N, D), jnp.float32),
                        pltpu.VMEM((N, DE), jnp.bfloat16),
                        pltpu.VMEM((8, N), jnp.float32)],
    )(x, W, a_src, a_dst, adj)


@jax.jit
def kernel(inputs, adjacency_matrix, W1, a_src1, a_dst1, W2, a_src2, a_dst2):
    x = _gat_layer(inputs, adjacency_matrix, W1, a_src1, a_dst1)
    x = _gat_layer(x, adjacency_matrix, W2, a_src2, a_dst2)
    return x
